# Initial kernel scaffold; baseline (speedup 1.0000x reference)
import jax
import jax.numpy as jnp
from jax.experimental import pallas as pl


def kernel(x_s, edge_index_s, batch_s, x_q, edge_index_q, batch_q, f_tasks_pos, W1, W2, Wg, Wc1, Wc2):
    raise NotImplementedError("write your pallas kernel here")


# speedup vs baseline: 3.5373x; 3.5373x over previous
import functools

import jax
import jax.numpy as jnp
from jax import lax
from jax.experimental import pallas as pl
from jax.experimental.pallas import tpu as pltpu
from jax.experimental.pallas import tpu_sc as plsc

N_NODES = 10000
EMB = 128
N_EDGES = 320000
K_SHOT = 10
N_WAY = 2
N_QUERY = 16
N_PROPERTY = 2
N_MEM_TASKS = 9
N_SUPPORT = N_WAY * K_SHOT

_NC, _NS, _L = 2, 16, 16
_NW = _NC * _NS

CH = 128
NCHUNK = 79
EPT = NCHUNK * CH
E_PAD = EPT * _NW
ACC_ROWS = 10240
RPT = ACC_ROWS // _NS

_sc_mesh = plsc.VectorSubcoreMesh(
    core_axis_name="c", subcore_axis_name="s", num_cores=_NC, num_subcores=_NS)


@functools.partial(
    pl.kernel,
    out_type=jax.ShapeDtypeStruct((_NC, ACC_ROWS, EMB), jnp.float32),
    mesh=_sc_mesh,
    scratch_types=[
        pltpu.VMEM((NCHUNK, CH), jnp.int32),
        pltpu.VMEM((NCHUNK, CH), jnp.int32),
        pltpu.VMEM((CH, EMB), jnp.float32),
        pltpu.VMEM_SHARED((ACC_ROWS, EMB), jnp.float32),
        pltpu.SemaphoreType.DMA,
    ],
)
def _sc_edge_scatter(x_hbm, src_hbm, dst_hbm, out_hbm,
                     src_v, dst_v, rows_v, acc_sh, sem):
    c = lax.axis_index("c")
    s = lax.axis_index("s")
    w = c * _NS + s

    zero = jnp.zeros((_L,), jnp.float32)

    def _zrow(r, carry):
        for k in range(EMB // _L):
            rows_v[r, pl.ds(k * _L, _L)] = zero
        return carry

    lax.fori_loop(0, CH, _zrow, 0)
    for b in range(RPT // CH):
        pltpu.sync_copy(rows_v, acc_sh.at[pl.ds(s * RPT + b * CH, CH)])
    plsc.subcore_barrier()

    pltpu.sync_copy(src_hbm.at[w], src_v)
    pltpu.sync_copy(dst_hbm.at[w], dst_v)

    def _chunk(j, carry):
        pltpu.async_copy(x_hbm.at[src_v.at[j]], rows_v, sem).wait()
        pltpu.sync_copy(rows_v, acc_sh.at[dst_v.at[j]], add=True)
        return carry

    lax.fori_loop(0, NCHUNK, _chunk, 0)
    plsc.subcore_barrier()

    pltpu.sync_copy(acc_sh.at[pl.ds(s * RPT, RPT)],
                    out_hbm.at[c, pl.ds(s * RPT, RPT)])


BLK = 400
NBLK = N_NODES // BLK
NG = 32


def _tc_fused_body(x_ref, p_ref, w_ref, b_ref, h_ref, pool_ref, cnt_ref):
    i = pl.program_id(0)
    t = x_ref[...] + p_ref[0] + p_ref[1]
    h = jnp.maximum(
        lax.dot(t, w_ref[...], preferred_element_type=jnp.float32), 0.0)
    h_ref[...] = h
    ids = b_ref[0, 0, :]
    gid = lax.broadcasted_iota(jnp.int32, (NG, BLK), 0)
    oh = (gid == ids[None, :]).astype(jnp.float32)
    psum = lax.dot(oh, h, preferred_element_type=jnp.float32,
                   precision=lax.Precision.HIGHEST)
    csum = jnp.broadcast_to(jnp.sum(oh, axis=1)[:, None], (NG, EMB))

    @pl.when(i == 0)
    def _():
        pool_ref[...] = jnp.zeros_like(pool_ref)
        cnt_ref[...] = jnp.zeros_like(cnt_ref)

    pool_ref[...] += psum
    cnt_ref[...] += csum


_tc_fused = pl.pallas_call(
    _tc_fused_body,
    grid=(NBLK,),
    in_specs=[
        pl.BlockSpec((BLK, EMB), lambda i: (i, 0)),
        pl.BlockSpec((_NC, BLK, EMB), lambda i: (0, i, 0)),
        pl.BlockSpec((EMB, EMB), lambda i: (0, 0)),
        pl.BlockSpec((1, 1, BLK), lambda i: (i, 0, 0)),
    ],
    out_specs=[
        pl.BlockSpec((BLK, EMB), lambda i: (i, 0)),
        pl.BlockSpec((NG, EMB), lambda i: (0, 0)),
        pl.BlockSpec((NG, EMB), lambda i: (0, 0)),
    ],
    out_shape=[
        jax.ShapeDtypeStruct((N_NODES, EMB), jnp.float32),
        jax.ShapeDtypeStruct((NG, EMB), jnp.float32),
        jax.ShapeDtypeStruct((NG, EMB), jnp.float32),
    ],
)


def _prep_edges(edge_index):
    pad = E_PAD - N_EDGES
    src = jnp.concatenate(
        [edge_index[0], jnp.zeros((pad,), jnp.int32)]).reshape(_NW, NCHUNK, CH)
    dump = (N_NODES + (jnp.arange(pad, dtype=jnp.int32)
                       % (ACC_ROWS - N_NODES)))
    dst = jnp.concatenate([edge_index[1], dump]).reshape(_NW, NCHUNK, CH)
    return src, dst


def _encode(x, edge_index, batch, n_graphs, W1, W2):
    src3, dst3 = _prep_edges(edge_index)
    batch3 = batch.reshape(NBLK, 1, BLK)
    mp1 = _sc_edge_scatter(x, src3, dst3)
    h1, pool1, cnt = _tc_fused(x, mp1, W1, batch3)
    mp2 = _sc_edge_scatter(h1, src3, dst3)
    _, pool2, _ = _tc_fused(h1, mp2, W2, batch3)
    cnt = jnp.maximum(cnt[:n_graphs, :1], 1.0)
    p1 = pool1[:n_graphs] / cnt
    p2 = pool2[:n_graphs] / cnt
    return p2, p1, p1 + p2


def _ftask_aug(s_emb, f_tasks_pos, Wg):
    neg = s_emb[:K_SHOT]
    pos = s_emb[K_SHOT:2 * K_SHOT]
    cur = jnp.mean(pos, axis=0, keepdims=True)
    ft = f_tasks_pos.reshape(N_MEM_TASKS, K_SHOT, EMB)
    allp = jnp.mean(ft, axis=1)
    num = jnp.sum(allp * cur, axis=1)
    den = jnp.linalg.norm(allp, axis=1) * jnp.linalg.norm(cur, axis=1) + 1e-8
    sim = num / den
    _, idx = lax.top_k(sim, N_PROPERTY)
    f_embs = jnp.take(ft, idx, axis=0).reshape(N_PROPERTY * K_SHOT, EMB)
    fc = jnp.concatenate([pos, f_embs], axis=0)
    xn = fc / (jnp.linalg.norm(fc, axis=-1, keepdims=True) + 1e-8)
    simm = xn @ xn.T
    new = jax.nn.relu(simm @ (fc @ Wg))
    return jnp.concatenate([neg, new[:K_SHOT]], axis=0)


def _tail(s_emb, q_emb, Wg, Wc1, Wc2):
    nq = q_emb.shape[0]
    m = jnp.concatenate(
        [jnp.broadcast_to(s_emb[None, :, :], (nq,) + s_emb.shape),
         q_emb[:, None, :]], axis=1)
    q, s, d = m.shape
    emb = m.reshape(q * s, d)
    xn = emb / (jnp.linalg.norm(emb, axis=-1, keepdims=True) + 1e-8)
    adj = xn @ xn.T
    new = jax.nn.relu(adj @ (emb @ Wg)).reshape(q, s, d)
    h = jax.nn.relu(new @ Wc1)
    logits = h @ Wc2
    return logits[:, :-1, :], logits[:, -1, :]


def kernel(x_s, edge_index_s, batch_s, x_q, edge_index_q, batch_q,
           f_tasks_pos, W1, W2, Wg, Wc1, Wc2):
    s_emb, sub_s_emb, cat_s_emb = _encode(
        x_s, edge_index_s, batch_s, N_SUPPORT, W1, W2)
    q_emb, sub_q_emb, cat_q_emb = _encode(
        x_q, edge_index_q, batch_q, N_QUERY, W1, W2)
    pos_emb = cat_s_emb[K_SHOT:2 * K_SHOT]
    s_emb = _ftask_aug(s_emb, f_tasks_pos, Wg)
    sub_s_emb = _ftask_aug(sub_s_emb, f_tasks_pos, Wg)
    cat_s_emb = _ftask_aug(cat_s_emb, f_tasks_pos, Wg)
    s_logits, q_logits = _tail(s_emb, q_emb, Wg, Wc1, Wc2)
    subs_logits, subq_logits = _tail(sub_s_emb, sub_q_emb, Wg, Wc1, Wc2)
    cats_logits, catq_logits = _tail(cat_s_emb, cat_q_emb, Wg, Wc1, Wc2)
    return (s_logits, q_logits, subs_logits, subq_logits,
            cats_logits, catq_logits, pos_emb)


# speedup vs baseline: 3.5423x; 1.0014x over previous
import functools

import jax
import jax.numpy as jnp
from jax import lax
from jax.experimental import pallas as pl
from jax.experimental.pallas import tpu as pltpu
from jax.experimental.pallas import tpu_sc as plsc

N_NODES = 10000
EMB = 128
N_EDGES = 320000
K_SHOT = 10
N_WAY = 2
N_QUERY = 16
N_PROPERTY = 2
N_MEM_TASKS = 9
N_SUPPORT = N_WAY * K_SHOT

_NC, _NS, _L = 2, 16, 16

CH = 128
NCHUNK = 160
SBLK = 16
NIB = NCHUNK // SBLK
EPT = NCHUNK * CH
E_PAD = EPT * _NS
ACC_ROWS = 10240
RPT = ACC_ROWS // _NS

_sc_mesh = plsc.VectorSubcoreMesh(
    core_axis_name="c", subcore_axis_name="s", num_cores=_NC, num_subcores=_NS)


@functools.partial(
    pl.kernel,
    out_type=jax.ShapeDtypeStruct((_NC, ACC_ROWS, EMB), jnp.float32),
    mesh=_sc_mesh,
    scratch_types=[
        pltpu.VMEM((SBLK, CH), jnp.int32),
        pltpu.VMEM((SBLK, CH), jnp.int32),
        pltpu.VMEM((CH, EMB), jnp.float32),
        pltpu.VMEM((CH, EMB), jnp.float32),
        pltpu.VMEM_SHARED((ACC_ROWS, EMB), jnp.float32),
        pltpu.SemaphoreType.DMA,
        pltpu.SemaphoreType.DMA,
        pltpu.SemaphoreType.DMA,
        pltpu.SemaphoreType.DMA,
    ],
)
def _sc_edge_scatter(x_hbm, src_hbm, dst_hbm, out_hbm,
                     src_v, dst_v, rows_a, rows_b, acc_sh,
                     sem_ga, sem_gb, sem_sa, sem_sb):
    c = lax.axis_index("c")
    s = lax.axis_index("s")

    zero = jnp.zeros((_L,), jnp.float32)

    def _zrow(r, carry):
        for k in range(EMB // _L):
            rows_a[r, pl.ds(k * _L, _L)] = zero
        return carry

    lax.fori_loop(0, CH, _zrow, 0)
    for b in range(RPT // CH):
        pltpu.sync_copy(rows_a, acc_sh.at[pl.ds(s * RPT + b * CH, CH)])
    plsc.subcore_barrier()

    def _gather(j, buf, sem):
        return pltpu.async_copy(x_hbm.at[src_v.at[j]], buf, sem)

    def _scatter(j, buf, sem):
        return pltpu.async_copy(buf, acc_sh.at[dst_v.at[j]], sem, add=True)

    def _wait_gather(j, buf, sem):
        pltpu.make_async_copy(x_hbm.at[src_v.at[j]], buf, sem).wait()

    def _wait_scatter(j, buf, sem):
        pltpu.make_async_copy(buf, acc_sh.at[dst_v.at[j]], sem).wait()

    def _block(b, carry):
        pltpu.sync_copy(src_hbm.at[c, s, pl.ds(b * SBLK, SBLK)], src_v)
        pltpu.sync_copy(dst_hbm.at[c, s, pl.ds(b * SBLK, SBLK)], dst_v)
        _gather(0, rows_a, sem_ga)

        def _pipe(i, carry):
            j0 = 2 * i
            j1 = 2 * i + 1
            _wait_gather(j0, rows_a, sem_ga)
            _scatter(j0, rows_a, sem_sa)

            @pl.when(i > 0)
            def _():
                _wait_scatter(j0 - 1, rows_b, sem_sb)

            _gather(j1, rows_b, sem_gb)
            _wait_gather(j1, rows_b, sem_gb)
            _scatter(j1, rows_b, sem_sb)

            @pl.when(i < SBLK // 2 - 1)
            def _():
                _wait_scatter(j0, rows_a, sem_sa)
                _gather(j0 + 2, rows_a, sem_ga)

            return carry

        lax.fori_loop(0, SBLK // 2, _pipe, 0)
        _wait_scatter(SBLK - 2, rows_a, sem_sa)
        _wait_scatter(SBLK - 1, rows_b, sem_sb)
        return carry

    lax.fori_loop(0, NIB, _block, 0)
    plsc.subcore_barrier()

    pltpu.sync_copy(acc_sh.at[pl.ds(s * RPT, RPT)],
                    out_hbm.at[c, pl.ds(s * RPT, RPT)])


BLK = 400
NBLK = N_NODES // BLK
NG = 64


def _tc_fused_body(x_ref, m_ref, w_ref, b_ref, h_ref, pool_ref, cnt_ref):
    i = pl.program_id(0)
    t = x_ref[...] + m_ref[0]
    h = jnp.maximum(
        lax.dot(t, w_ref[...], preferred_element_type=jnp.float32), 0.0)
    h_ref[...] = h
    ids = b_ref[0, 0, :] + (i // NBLK) * 32
    gid = lax.broadcasted_iota(jnp.int32, (NG, BLK), 0)
    oh = (gid == ids[None, :]).astype(jnp.float32)
    psum = lax.dot(oh, h, preferred_element_type=jnp.float32,
                   precision=lax.Precision.HIGHEST)
    csum = jnp.broadcast_to(jnp.sum(oh, axis=1)[:, None], (NG, EMB))

    @pl.when(i == 0)
    def _():
        pool_ref[...] = jnp.zeros_like(pool_ref)
        cnt_ref[...] = jnp.zeros_like(cnt_ref)

    pool_ref[...] += psum
    cnt_ref[...] += csum


_tc_fused = pl.pallas_call(
    _tc_fused_body,
    grid=(2 * NBLK,),
    in_specs=[
        pl.BlockSpec((BLK, EMB), lambda i: (i, 0)),
        pl.BlockSpec((1, BLK, EMB), lambda i: (i // NBLK, i % NBLK, 0)),
        pl.BlockSpec((EMB, EMB), lambda i: (0, 0)),
        pl.BlockSpec((1, 1, BLK), lambda i: (i, 0, 0)),
    ],
    out_specs=[
        pl.BlockSpec((BLK, EMB), lambda i: (i, 0)),
        pl.BlockSpec((NG, EMB), lambda i: (0, 0)),
        pl.BlockSpec((NG, EMB), lambda i: (0, 0)),
    ],
    out_shape=[
        jax.ShapeDtypeStruct((2 * N_NODES, EMB), jnp.float32),
        jax.ShapeDtypeStruct((NG, EMB), jnp.float32),
        jax.ShapeDtypeStruct((NG, EMB), jnp.float32),
    ],
)


def _prep_edges(edge_index, node_offset):
    pad = E_PAD - N_EDGES
    src = jnp.concatenate(
        [edge_index[0] + jnp.int32(node_offset),
         jnp.full((pad,), node_offset, jnp.int32)]).reshape(_NS, NCHUNK, CH)
    dump = (N_NODES + (jnp.arange(pad, dtype=jnp.int32)
                       % (ACC_ROWS - N_NODES)))
    dst = jnp.concatenate([edge_index[1], dump]).reshape(_NS, NCHUNK, CH)
    return src, dst


def _encode_both(x_s, edge_index_s, batch_s, x_q, edge_index_q, batch_q,
                 W1, W2):
    src_s, dst_s = _prep_edges(edge_index_s, 0)
    src_q, dst_q = _prep_edges(edge_index_q, N_NODES)
    src3 = jnp.stack([src_s, src_q])
    dst3 = jnp.stack([dst_s, dst_q])
    batch3 = jnp.concatenate([batch_s, batch_q]).reshape(2 * NBLK, 1, BLK)
    x = jnp.concatenate([x_s, x_q], axis=0)

    m1 = _sc_edge_scatter(x, src3, dst3)
    h1, pool1, cnt = _tc_fused(x, m1, W1, batch3)
    m2 = _sc_edge_scatter(h1, src3, dst3)
    _, pool2, _ = _tc_fused(h1, m2, W2, batch3)

    def _finish(lo, n_graphs):
        c = jnp.maximum(cnt[lo:lo + n_graphs, :1], 1.0)
        p1 = pool1[lo:lo + n_graphs] / c
        p2 = pool2[lo:lo + n_graphs] / c
        return p2, p1, p1 + p2

    return _finish(0, N_SUPPORT), _finish(32, N_QUERY)


def _ftask_aug(s_emb, f_tasks_pos, Wg):
    neg = s_emb[:K_SHOT]
    pos = s_emb[K_SHOT:2 * K_SHOT]
    cur = jnp.mean(pos, axis=0, keepdims=True)
    ft = f_tasks_pos.reshape(N_MEM_TASKS, K_SHOT, EMB)
    allp = jnp.mean(ft, axis=1)
    num = jnp.sum(allp * cur, axis=1)
    den = jnp.linalg.norm(allp, axis=1) * jnp.linalg.norm(cur, axis=1) + 1e-8
    sim = num / den
    _, idx = lax.top_k(sim, N_PROPERTY)
    f_embs = jnp.take(ft, idx, axis=0).reshape(N_PROPERTY * K_SHOT, EMB)
    fc = jnp.concatenate([pos, f_embs], axis=0)
    xn = fc / (jnp.linalg.norm(fc, axis=-1, keepdims=True) + 1e-8)
    simm = xn @ xn.T
    new = jax.nn.relu(simm @ (fc @ Wg))
    return jnp.concatenate([neg, new[:K_SHOT]], axis=0)


def _tail(s_emb, q_emb, Wg, Wc1, Wc2):
    nq = q_emb.shape[0]
    m = jnp.concatenate(
        [jnp.broadcast_to(s_emb[None, :, :], (nq,) + s_emb.shape),
         q_emb[:, None, :]], axis=1)
    q, s, d = m.shape
    emb = m.reshape(q * s, d)
    xn = emb / (jnp.linalg.norm(emb, axis=-1, keepdims=True) + 1e-8)
    adj = xn @ xn.T
    new = jax.nn.relu(adj @ (emb @ Wg)).reshape(q, s, d)
    h = jax.nn.relu(new @ Wc1)
    logits = h @ Wc2
    return logits[:, :-1, :], logits[:, -1, :]


def kernel(x_s, edge_index_s, batch_s, x_q, edge_index_q, batch_q,
           f_tasks_pos, W1, W2, Wg, Wc1, Wc2):
    (s_emb, sub_s_emb, cat_s_emb), (q_emb, sub_q_emb, cat_q_emb) = (
        _encode_both(x_s, edge_index_s, batch_s,
                     x_q, edge_index_q, batch_q, W1, W2))
    pos_emb = cat_s_emb[K_SHOT:2 * K_SHOT]
    s_emb = _ftask_aug(s_emb, f_tasks_pos, Wg)
    sub_s_emb = _ftask_aug(sub_s_emb, f_tasks_pos, Wg)
    cat_s_emb = _ftask_aug(cat_s_emb, f_tasks_pos, Wg)
    s_logits, q_logits = _tail(s_emb, q_emb, Wg, Wc1, Wc2)
    subs_logits, subq_logits = _tail(sub_s_emb, sub_q_emb, Wg, Wc1, Wc2)
    cats_logits, catq_logits = _tail(cat_s_emb, cat_q_emb, Wg, Wc1, Wc2)
    return (s_logits, q_logits, subs_logits, subq_logits,
            cats_logits, catq_logits, pos_emb)
